# baseline (speedup 1.0000x reference)
import functools

import jax
import jax.numpy as jnp
from jax.experimental import pallas as pl


def _loss_kernel(q_ref, items_ref, out_ref, *, block_t: int, chunk_t: int,
                 m: int, inv_tc: float):
    i = pl.program_id(0)

    items = items_ref[...]
    n2 = jnp.sum(items * items, axis=1)
    qall = q_ref[...]
    partial = jnp.sum(qall * qall)
    for k in range(block_t // chunk_t):
        q = q_ref[pl.ds(k * chunk_t, chunk_t), :]
        score = jax.lax.dot_general(
            q, items, (((1,), (1,)), ((), ())),
            preferred_element_type=jnp.float32,
        )
        rowmax = jnp.max(score, axis=1, keepdims=True)
        is_max = score == rowmax
        n2b = jnp.broadcast_to(n2[None, :], score.shape)
        n2_at = jnp.sum(jnp.where(is_max, n2b, 0.0), axis=1)
        partial += jnp.sum(n2_at) - 2.0 * jnp.sum(rowmax)

    @pl.when(i == 0)
    def _():
        out_ref[...] = jnp.zeros_like(out_ref)

    out_ref[...] += jnp.reshape(partial * inv_tc, (1, 1))


@jax.jit
def kernel(queries, items):
    n, l, c = queries.shape
    m = items.shape[0]
    t = n * l
    q = queries.reshape(t, c)

    block_t = 4096
    chunk_t = 1024
    grid = (t // block_t,)

    total = pl.pallas_call(
        functools.partial(_loss_kernel, block_t=block_t, chunk_t=chunk_t,
                          m=m, inv_tc=1.0 / (t * c)),
        grid=grid,
        in_specs=[
            pl.BlockSpec((block_t, c), lambda i: (i, 0)),
            pl.BlockSpec((m, c), lambda i: (0, 0)),
        ],
        out_specs=pl.BlockSpec((1, 1), lambda i: (0, 0)),
        out_shape=jax.ShapeDtypeStruct((1, 1), jnp.float32),
    )(q, items)

    return total[0, 0]


# speedup vs baseline: 1.1482x; 1.1482x over previous
import functools

import jax
import jax.numpy as jnp
from jax.experimental import pallas as pl


def _loss_kernel(q_ref, items_ref, out_ref, *, block_t: int, chunk_t: int,
                 m: int, inv_tc: float):
    i = pl.program_id(0)

    items = items_ref[...]
    n2 = jnp.sum(items * items, axis=1)
    partial = jnp.zeros((), jnp.float32)
    for k in range(block_t // chunk_t):
        q = q_ref[pl.ds(k * chunk_t, chunk_t), :]
        score = jax.lax.dot_general(
            q, items, (((1,), (1,)), ((), ())),
            preferred_element_type=jnp.float32,
        )
        rowmax = jnp.max(score, axis=1, keepdims=True)
        is_max = score == rowmax
        n2b = jnp.broadcast_to(n2[None, :], score.shape)
        n2_at = jnp.sum(jnp.where(is_max, n2b, 0.0), axis=1)
        partial += (
            jnp.sum(q * q)
            + jnp.sum(n2_at)
            - 2.0 * jnp.sum(rowmax)
        )

    @pl.when(i == 0)
    def _():
        out_ref[...] = jnp.zeros_like(out_ref)

    out_ref[...] += jnp.reshape(partial * inv_tc, (1, 1))


@jax.jit
def kernel(queries, items):
    n, l, c = queries.shape
    m = items.shape[0]
    t = n * l
    q = queries.reshape(t, c)

    block_t = 4096
    chunk_t = 1024
    grid = (t // block_t,)

    total = pl.pallas_call(
        functools.partial(_loss_kernel, block_t=block_t, chunk_t=chunk_t,
                          m=m, inv_tc=1.0 / (t * c)),
        grid=grid,
        in_specs=[
            pl.BlockSpec((block_t, c), lambda i: (i, 0)),
            pl.BlockSpec((m, c), lambda i: (0, 0)),
        ],
        out_specs=pl.BlockSpec((1, 1), lambda i: (0, 0)),
        out_shape=jax.ShapeDtypeStruct((1, 1), jnp.float32),
    )(q, items)

    return total[0, 0]
